# Initial kernel scaffold; baseline (speedup 1.0000x reference)
import functools

import jax
import jax.numpy as jnp
from jax import lax
from jax.experimental import pallas as pl
from jax.experimental.pallas import tpu as pltpu
from jax.experimental.pallas import tpu_sc as plsc

NUM_SYMBOLS = 1000000
EMBED_DIM = 64
BATCH = 16384
HIST = 50

NC = 2
NS = 16
NW = NC * NS
B = BATCH * HIST
BPW = B // NW
CH = 128
NCH = BPW // CH
NBUF = 8
LOOK = 4

_mesh = plsc.VectorSubcoreMesh(core_axis_name="c", subcore_axis_name="s")


@functools.partial(
    pl.kernel,
    mesh=_mesh,
    out_type=jax.ShapeDtypeStruct((NW, NCH, CH, EMBED_DIM), jnp.float32),
    scratch_types=[
        pltpu.VMEM((NCH, CH), jnp.int32),
        pltpu.VMEM((NBUF, CH, EMBED_DIM), jnp.float32),
        pltpu.SemaphoreType.DMA((NBUF,)),
        pltpu.SemaphoreType.DMA((NBUF,)),
    ],
)
def _emb_lookup(x_hbm, tbl_hbm, out_hbm, idx_v, rows_v, gsem, wsem):
    wid = lax.axis_index("s") * NC + lax.axis_index("c")
    pltpu.sync_copy(x_hbm.at[wid], idx_v)

    for b in range(LOOK):
        pltpu.async_copy(tbl_hbm.at[idx_v.at[b]], rows_v.at[b], gsem.at[b])

    def group(go, carry):
        for b in range(NBUF):
            g = go * NBUF + b
            pltpu.make_async_copy(
                tbl_hbm.at[idx_v.at[g]], rows_v.at[b], gsem.at[b]
            ).wait()
            pltpu.async_copy(rows_v.at[b], out_hbm.at[wid].at[g], wsem.at[b])
            s2 = (b + LOOK) % NBUF
            g2 = g + LOOK

            @pl.when(g2 >= NBUF)
            def _():
                pltpu.make_async_copy(
                    rows_v.at[s2], out_hbm.at[wid].at[g2 - NBUF], wsem.at[s2]
                ).wait()

            @pl.when(g2 < NCH)
            def _():
                pltpu.async_copy(
                    tbl_hbm.at[idx_v.at[g2]], rows_v.at[s2], gsem.at[s2]
                )
        return carry

    lax.fori_loop(0, NCH // NBUF, group, 0)

    for i in range(LOOK):
        g = NCH - LOOK + i
        b = g % NBUF
        pltpu.make_async_copy(
            rows_v.at[b], out_hbm.at[wid].at[g], wsem.at[b]
        ).wait()


def kernel(x, table):
    xf = x.reshape(NW, NCH, CH).astype(jnp.int32)
    out = _emb_lookup(xf, table)
    return out.reshape(BATCH, HIST, EMBED_DIM)


# speedup vs baseline: 1.8762x; 1.8762x over previous
import functools

import jax
import jax.numpy as jnp
from jax import lax
from jax.experimental import pallas as pl
from jax.experimental.pallas import tpu as pltpu
from jax.experimental.pallas import tpu_sc as plsc

NUM_SYMBOLS = 1000000
EMBED_DIM = 64
BATCH = 16384
HIST = 50

NC = 2
NS = 16
NW = NC * NS
B = BATCH * HIST
BPW = B // NW
CH = 128
NCH = BPW // CH
NBUF = 8
LOOK = 4

_mesh = plsc.VectorSubcoreMesh(core_axis_name="c", subcore_axis_name="s")


@functools.partial(
    pl.kernel,
    mesh=_mesh,
    out_type=jax.ShapeDtypeStruct((NW, NCH, CH, EMBED_DIM), jnp.float32),
    compiler_params=pltpu.CompilerParams(use_tc_tiling_on_sc=False),
    scratch_types=[
        pltpu.VMEM((NCH, CH), jnp.int32),
        pltpu.VMEM((NBUF, CH, EMBED_DIM), jnp.float32),
        pltpu.SemaphoreType.DMA((NBUF,)),
        pltpu.SemaphoreType.DMA((NBUF,)),
    ],
)
def _emb_lookup(x_hbm, tbl_hbm, out_hbm, idx_v, rows_v, gsem, wsem):
    wid = lax.axis_index("s") * NC + lax.axis_index("c")
    pltpu.sync_copy(x_hbm.at[wid], idx_v)

    for b in range(LOOK):
        pltpu.async_copy(tbl_hbm.at[idx_v.at[b]], rows_v.at[b], gsem.at[b])

    def group(go, carry):
        for b in range(NBUF):
            g = go * NBUF + b
            pltpu.make_async_copy(
                tbl_hbm.at[idx_v.at[g]], rows_v.at[b], gsem.at[b]
            ).wait()
            pltpu.async_copy(rows_v.at[b], out_hbm.at[wid].at[g], wsem.at[b])
            s2 = (b + LOOK) % NBUF
            g2 = g + LOOK

            @pl.when(g2 >= NBUF)
            def _():
                pltpu.make_async_copy(
                    rows_v.at[s2], out_hbm.at[wid].at[g2 - NBUF], wsem.at[s2]
                ).wait()

            @pl.when(g2 < NCH)
            def _():
                pltpu.async_copy(
                    tbl_hbm.at[idx_v.at[g2]], rows_v.at[s2], gsem.at[s2]
                )
        return carry

    lax.fori_loop(0, NCH // NBUF, group, 0)

    for i in range(LOOK):
        g = NCH - LOOK + i
        b = g % NBUF
        pltpu.make_async_copy(
            rows_v.at[b], out_hbm.at[wid].at[g], wsem.at[b]
        ).wait()


def kernel(x, table):
    xf = x.reshape(NW, NCH, CH).astype(jnp.int32)
    out = _emb_lookup(xf, table)
    return out.reshape(BATCH, HIST, EMBED_DIM)


# speedup vs baseline: 1.9664x; 1.0480x over previous
import functools

import jax
import jax.numpy as jnp
from jax import lax
from jax.experimental import pallas as pl
from jax.experimental.pallas import tpu as pltpu
from jax.experimental.pallas import tpu_sc as plsc

NUM_SYMBOLS = 1000000
EMBED_DIM = 64
BATCH = 16384
HIST = 50

NC = 2
NS = 16
NW = NC * NS
CH = 128
CPW = BATCH // CH // NW
NBLK = HIST * CPW
NBUF = 8
LOOK = 4

_mesh = plsc.VectorSubcoreMesh(core_axis_name="c", subcore_axis_name="s")


@functools.partial(
    pl.kernel,
    mesh=_mesh,
    out_type=jax.ShapeDtypeStruct((HIST, BATCH, EMBED_DIM), jnp.float32),
    compiler_params=pltpu.CompilerParams(use_tc_tiling_on_sc=False),
    scratch_types=[
        pltpu.VMEM((HIST, CPW * CH), jnp.int32),
        pltpu.VMEM((NBUF, CH, EMBED_DIM), jnp.float32),
        pltpu.SemaphoreType.DMA((NBUF,)),
        pltpu.SemaphoreType.DMA((NBUF,)),
    ],
)
def _emb_lookup(xt_hbm, tbl_hbm, out_hbm, idx_v, rows_v, gsem, wsem):
    wid = lax.axis_index("s") * NC + lax.axis_index("c")
    pltpu.sync_copy(xt_hbm.at[:, pl.ds(wid * (CPW * CH), CPW * CH)], idx_v)

    def fire_gather(g, slot):
        h = g // CPW
        cg = g % CPW
        pltpu.async_copy(
            tbl_hbm.at[idx_v.at[h, pl.ds(cg * CH, CH)]],
            rows_v.at[slot],
            gsem.at[slot],
        )

    def wait_gather(slot):
        pltpu.make_async_copy(
            tbl_hbm.at[idx_v.at[0, pl.ds(0, CH)]],
            rows_v.at[slot],
            gsem.at[slot],
        ).wait()

    def out_dst(g):
        h = g // CPW
        b0 = (wid * CPW + (g % CPW)) * CH
        return out_hbm.at[h, pl.ds(b0, CH), :]

    def fire_write(g, slot):
        pltpu.async_copy(rows_v.at[slot], out_dst(g), wsem.at[slot])

    def wait_write(g, slot):
        pltpu.make_async_copy(rows_v.at[slot], out_dst(g), wsem.at[slot]).wait()

    for b in range(LOOK):
        fire_gather(b, b)

    def group(go, carry):
        for b in range(NBUF):
            g = go * NBUF + b
            wait_gather(b)
            fire_write(g, b)
            s2 = (b + LOOK) % NBUF
            g2 = g + LOOK

            @pl.when(g2 >= NBUF)
            def _():
                wait_write(g2 - NBUF, s2)

            @pl.when(g2 < NBLK)
            def _():
                fire_gather(g2, s2)
        return carry

    lax.fori_loop(0, NBLK // NBUF, group, 0)

    for i in range(LOOK):
        g = NBLK - LOOK + i
        wait_write(g, g % NBUF)


def kernel(x, table):
    xt = x.T.astype(jnp.int32)
    out_t = _emb_lookup(xt, table)
    return out_t.transpose(1, 0, 2)


# speedup vs baseline: 1.9720x; 1.0028x over previous
import functools

import jax
import jax.numpy as jnp
from jax import lax
from jax.experimental import pallas as pl
from jax.experimental.pallas import tpu as pltpu
from jax.experimental.pallas import tpu_sc as plsc

NUM_SYMBOLS = 1000000
EMBED_DIM = 64
BATCH = 16384
HIST = 50

NC = 2
NS = 16
NW = NC * NS
CH = 128
HCH = CH // 2
CPW = BATCH // CH // NW
NBLK = HIST * CPW
NBUF = 8
LOOK = 4

_mesh = plsc.VectorSubcoreMesh(core_axis_name="c", subcore_axis_name="s")


@functools.partial(
    pl.kernel,
    mesh=_mesh,
    out_type=jax.ShapeDtypeStruct((HIST, BATCH // 2, 2 * EMBED_DIM), jnp.float32),
    compiler_params=pltpu.CompilerParams(use_tc_tiling_on_sc=False),
    scratch_types=[
        pltpu.VMEM((HIST, CPW * CH), jnp.int32),
        pltpu.VMEM((NBUF, 2, HCH, EMBED_DIM), jnp.float32),
        pltpu.SemaphoreType.DMA((NBUF,)),
        pltpu.SemaphoreType.DMA((NBUF,)),
    ],
)
def _emb_lookup(xr_hbm, tbl_hbm, out_hbm, idx_v, rows_v, gsem, wsem):
    wid = lax.axis_index("s") * NC + lax.axis_index("c")
    pltpu.sync_copy(xr_hbm.at[:, pl.ds(wid * (CPW * CH), CPW * CH)], idx_v)

    def fire_gather(g, slot):
        h = g // CPW
        cg = g % CPW
        for half in range(2):
            pltpu.async_copy(
                tbl_hbm.at[idx_v.at[h, pl.ds(cg * CH + half * HCH, HCH)]],
                rows_v.at[slot, half],
                gsem.at[slot],
            )

    def wait_gather(slot):
        for half in range(2):
            pltpu.make_async_copy(
                tbl_hbm.at[idx_v.at[0, pl.ds(0, HCH)]],
                rows_v.at[slot, half],
                gsem.at[slot],
            ).wait()

    def write_parts(g, slot, fire):
        h = g // CPW
        p0 = (wid * CPW + (g % CPW)) * HCH
        for half in range(2):
            dst = out_hbm.at[
                h, pl.ds(p0, HCH), pl.ds(half * EMBED_DIM, EMBED_DIM)
            ]
            cp = (
                pltpu.async_copy(rows_v.at[slot, half], dst, wsem.at[slot])
                if fire
                else pltpu.make_async_copy(
                    rows_v.at[slot, half], dst, wsem.at[slot]
                ).wait()
            )

    for b in range(LOOK):
        fire_gather(b, b)

    def group(go, carry):
        for b in range(NBUF):
            g = go * NBUF + b
            wait_gather(b)
            write_parts(g, b, True)
            s2 = (b + LOOK) % NBUF
            g2 = g + LOOK

            @pl.when(g2 >= NBUF)
            def _():
                write_parts(g2 - NBUF, s2, False)

            @pl.when(g2 < NBLK)
            def _():
                fire_gather(g2, s2)
        return carry

    lax.fori_loop(0, NBLK // NBUF, group, 0)

    for i in range(LOOK):
        g = NBLK - LOOK + i
        write_parts(g, g % NBUF, False)


def kernel(x, table):
    xr = (
        x.T.astype(jnp.int32)
        .reshape(HIST, BATCH // CH, HCH, 2)
        .transpose(0, 1, 3, 2)
        .reshape(HIST, BATCH)
    )
    out_pair = _emb_lookup(xr, table)
    return (
        out_pair.reshape(HIST, BATCH // 2, 2, EMBED_DIM)
        .transpose(1, 2, 0, 3)
        .reshape(BATCH, HIST, EMBED_DIM)
    )
